# Initial kernel scaffold; baseline (speedup 1.0000x reference)
import jax
import jax.numpy as jnp
from jax.experimental import pallas as pl


def kernel(agent_position, agent_heading, agent_valid_mask, map_polygon_center, map_valid_mask, pos_table_0, pos_table_1, head_table_0, head_table_1, w1, ln1_g, ln1_b, w2, ln2_g, ln2_b, w3, b3, oob_w, window_T):
    raise NotImplementedError("write your pallas kernel here")


# speedup vs baseline: 3.4689x; 3.4689x over previous
import functools

import jax
import jax.numpy as jnp
from jax import lax
from jax.experimental import pallas as pl
from jax.experimental.pallas import tpu as pltpu
from jax.experimental.pallas import tpu_sc as plsc


def _sc_features(posx, posy, heading, t0, t1, h0, h1):
    tokens = posx.shape[0]
    info = plsc.get_sparse_core_info()
    ncores, nsub = info.num_cores, info.num_subcores
    nw = ncores * nsub
    tpw = tokens // nw
    assert tpw * nw == tokens
    ch = 1
    for cand in range(16, 513, 16):
        if tpw % cand == 0:
            ch = cand
    n_chunks = tpw // ch
    n_groups = ch // 16

    n0, d0 = t0.shape
    n1, d1 = t1.shape
    nh0, dh = h0.shape
    nh1, _ = h1.shape

    mesh = plsc.VectorSubcoreMesh(core_axis_name="c", subcore_axis_name="s")

    @functools.partial(
        pl.kernel,
        out_type=jax.ShapeDtypeStruct((tokens * 128,), jnp.float32),
        mesh=mesh,
        compiler_params=pltpu.CompilerParams(needs_layout_passes=False),
        scratch_types=[
            pltpu.VMEM((n0 * d0,), jnp.float32),
            pltpu.VMEM((n1 * d1,), jnp.float32),
            pltpu.VMEM((nh0 * dh,), jnp.float32),
            pltpu.VMEM((nh1 * dh,), jnp.float32),
            pltpu.VMEM((ch,), jnp.float32),
            pltpu.VMEM((ch,), jnp.float32),
            pltpu.VMEM((ch,), jnp.float32),
            pltpu.VMEM((ch * 128,), jnp.float32),
        ],
    )
    def sc_kernel(px_h, py_h, hd_h, t0_h, t1_h, h0_h, h1_h, out_h,
                  t0_v, t1_v, h0_v, h1_v, px_v, py_v, hd_v, xb_v):
        wid = lax.axis_index("s") * ncores + lax.axis_index("c")
        base = wid * tpw
        pltpu.sync_copy(t0_h, t0_v)
        pltpu.sync_copy(t1_h, t1_v)
        pltpu.sync_copy(h0_h, h0_v)
        pltpu.sync_copy(h1_h, h1_v)
        lane = lax.iota(jnp.int32, 16)
        zeros16 = jnp.zeros((16,), jnp.float32)

        def chunk_body(c, carry):
            tok0 = base + c * ch
            pltpu.sync_copy(px_h.at[pl.ds(tok0, ch)], px_v)
            pltpu.sync_copy(py_h.at[pl.ds(tok0, ch)], py_v)
            pltpu.sync_copy(hd_h.at[pl.ds(tok0, ch)], hd_v)

            def group_body(g, carry2):
                s = g * 16
                px = px_v[pl.ds(s, 16)]
                py = py_v[pl.ds(s, 16)]
                hd = hd_v[pl.ds(s, 16)]
                ex = px + 300.0
                ix0 = jnp.clip(ex.astype(jnp.int32), 0, n0 - 1)
                rx = ex - ix0.astype(jnp.float32)
                ix1 = jnp.clip((rx / 0.01).astype(jnp.int32), 0, n1 - 1)
                ey = py + 300.0
                iy0 = jnp.clip(ey.astype(jnp.int32), 0, n0 - 1)
                ry = ey - iy0.astype(jnp.float32)
                iy1 = jnp.clip((ry / 0.01).astype(jnp.int32), 0, n1 - 1)
                eh = hd * 180.0 / jnp.pi + 180.0
                ih0 = jnp.clip((eh / 20.0).astype(jnp.int32), 0, nh0 - 1)
                rh = eh - ih0.astype(jnp.float32) * 20.0
                ih1 = jnp.clip(rh.astype(jnp.int32), 0, nh1 - 1)

                b128 = (s + lane) * 128
                gx0 = ix0 * d0
                gx1 = ix1 * d1
                gy0 = iy0 * d0
                gy1 = iy1 * d1
                gh0 = ih0 * dh
                gh1 = ih1 * dh
                for j in range(d0):
                    v = plsc.load_gather(t0_v, [gx0 + j])
                    plsc.store_scatter(xb_v, [b128 + j], v)
                for j in range(d1):
                    v = plsc.load_gather(t1_v, [gx1 + j])
                    plsc.store_scatter(xb_v, [b128 + (d0 + j)], v)
                for j in range(d0):
                    v = plsc.load_gather(t0_v, [gy0 + j])
                    plsc.store_scatter(xb_v, [b128 + (d0 + d1 + j)], v)
                for j in range(d1):
                    v = plsc.load_gather(t1_v, [gy1 + j])
                    plsc.store_scatter(xb_v, [b128 + (2 * d0 + d1 + j)], v)
                off_h = 2 * (d0 + d1)
                for j in range(dh):
                    v = plsc.load_gather(h0_v, [gh0 + j])
                    plsc.store_scatter(xb_v, [b128 + (off_h + j)], v)
                for j in range(dh):
                    v = plsc.load_gather(h1_v, [gh1 + j])
                    plsc.store_scatter(xb_v, [b128 + (off_h + dh + j)], v)
                for j in range(off_h + 2 * dh, 128):
                    plsc.store_scatter(xb_v, [b128 + j], zeros16)
                return carry2

            lax.fori_loop(0, n_groups, group_body, 0)
            pltpu.sync_copy(xb_v, out_h.at[pl.ds(tok0 * 128, ch * 128)])
            return carry

        lax.fori_loop(0, n_chunks, chunk_body, 0)

    return sc_kernel(posx, posy, heading,
                     t0.reshape(-1), t1.reshape(-1),
                     h0.reshape(-1), h1.reshape(-1))


def _ln(x, g, b, eps=1e-5):
    mu = jnp.mean(x, axis=-1, keepdims=True)
    var = jnp.mean((x - mu) ** 2, axis=-1, keepdims=True)
    return (x - mu) * lax.rsqrt(var + eps) * g + b


def _tc_mlp(x2d, maskf, w1p, g1, b1, w2, g2, b2, w3, b3, oob):
    tokens = x2d.shape[0]
    tile = 256
    grid = tokens // tile
    assert grid * tile == tokens

    def body(x_ref, m_ref, w1_ref, g1_ref, b1_ref, w2_ref, g2_ref, b2_ref,
             w3_ref, b3_ref, oob_ref, o_ref):
        x = x_ref[...]
        h = jnp.dot(x, w1_ref[...], preferred_element_type=jnp.float32)
        h = jnp.maximum(_ln(h, g1_ref[...], b1_ref[...]), 0.0)
        h = jnp.dot(h, w2_ref[...], preferred_element_type=jnp.float32)
        h = jnp.maximum(_ln(h, g2_ref[...], b2_ref[...]), 0.0)
        y = jnp.dot(h, w3_ref[...], preferred_element_type=jnp.float32)
        y = y + b3_ref[...]
        m = m_ref[...]
        o_ref[...] = jnp.where(m > 0.0, y, oob_ref[...])

    full = lambda shape: pl.BlockSpec(shape, lambda i: (0, 0))
    return pl.pallas_call(
        body,
        grid=(grid,),
        in_specs=[
            pl.BlockSpec((tile, 128), lambda i: (i, 0)),
            pl.BlockSpec((tile, 1), lambda i: (i, 0)),
            full((128, 256)),
            full((1, 256)),
            full((1, 256)),
            full((256, 256)),
            full((1, 256)),
            full((1, 256)),
            full((256, 256)),
            full((1, 256)),
            full((1, 256)),
        ],
        out_specs=pl.BlockSpec((tile, 256), lambda i: (i, 0)),
        out_shape=jax.ShapeDtypeStruct((tokens, 256), jnp.float32),
    )(x2d, maskf, w1p, g1, b1, w2, g2, b2, w3, b3, oob)


def kernel(agent_position, agent_heading, agent_valid_mask, map_polygon_center,
           map_valid_mask, pos_table_0, pos_table_1, head_table_0, head_table_1,
           w1, ln1_g, ln1_b, w2, ln2_g, ln2_b, w3, b3, oob_w, window_T):
    B, N, T = agent_heading.shape
    tokens = B * (T - 1) * N

    posx = jnp.swapaxes(agent_position[:, :, 1:, 0], 1, 2).reshape(-1)
    posy = jnp.swapaxes(agent_position[:, :, 1:, 1], 1, 2).reshape(-1)
    hd = jnp.swapaxes(agent_heading[:, :, 1:], 1, 2).reshape(-1)
    maskf = jnp.swapaxes(agent_valid_mask[:, :, 1:], 1, 2).reshape(-1, 1)
    maskf = maskf.astype(jnp.float32)

    x = _sc_features(posx, posy, hd, pos_table_0, pos_table_1,
                     head_table_0, head_table_1)
    x2d = x.reshape(tokens, 128)

    w1p = jnp.pad(w1, ((0, 128 - w1.shape[0]), (0, 0)))
    out = _tc_mlp(x2d, maskf, w1p,
                  ln1_g.reshape(1, -1), ln1_b.reshape(1, -1),
                  w2, ln2_g.reshape(1, -1), ln2_b.reshape(1, -1),
                  w3, b3.reshape(1, -1), oob_w)
    return out.reshape(B, T - 1, N, 256)


# speedup vs baseline: 5.5590x; 1.6025x over previous
import functools

import jax
import jax.numpy as jnp
from jax import lax
from jax.experimental import pallas as pl
from jax.experimental.pallas import tpu as pltpu
from jax.experimental.pallas import tpu_sc as plsc


def _sc_features(posx, posy, heading, t0, t1, h0, h1):
    tokens = posx.shape[0]
    info = plsc.get_sparse_core_info()
    ncores, nsub = info.num_cores, info.num_subcores
    nw = ncores * nsub
    tpw = tokens // nw
    assert tpw * nw == tokens
    ch = 1
    for cand in range(16, 513, 16):
        if tpw % cand == 0:
            ch = cand
    n_chunks = tpw // ch
    n_groups = ch // 16

    n0, d0 = t0.shape
    n1, d1 = t1.shape
    nh0, dh = h0.shape
    nh1, _ = h1.shape

    mesh = plsc.VectorSubcoreMesh(core_axis_name="c", subcore_axis_name="s")

    @functools.partial(
        pl.kernel,
        out_type=jax.ShapeDtypeStruct((tokens * 128,), jnp.float32),
        mesh=mesh,
        compiler_params=pltpu.CompilerParams(needs_layout_passes=False),
        scratch_types=[
            pltpu.VMEM((n0 * d0,), jnp.float32),
            pltpu.VMEM((n1 * d1,), jnp.float32),
            pltpu.VMEM((nh0 * dh,), jnp.float32),
            pltpu.VMEM((nh1 * dh,), jnp.float32),
            pltpu.VMEM((ch,), jnp.float32),
            pltpu.VMEM((ch,), jnp.float32),
            pltpu.VMEM((ch,), jnp.float32),
            pltpu.VMEM((ch * 128,), jnp.float32),
        ],
    )
    def sc_kernel(px_h, py_h, hd_h, t0_h, t1_h, h0_h, h1_h, out_h,
                  t0_v, t1_v, h0_v, h1_v, px_v, py_v, hd_v, xb_v):
        wid = lax.axis_index("s") * ncores + lax.axis_index("c")
        base = wid * tpw
        pltpu.sync_copy(t0_h, t0_v)
        pltpu.sync_copy(t1_h, t1_v)
        pltpu.sync_copy(h0_h, h0_v)
        pltpu.sync_copy(h1_h, h1_v)
        lane = lax.iota(jnp.int32, 16)
        zeros16 = jnp.zeros((16,), jnp.float32)

        def chunk_body(c, carry):
            tok0 = base + c * ch
            pltpu.sync_copy(px_h.at[pl.ds(tok0, ch)], px_v)
            pltpu.sync_copy(py_h.at[pl.ds(tok0, ch)], py_v)
            pltpu.sync_copy(hd_h.at[pl.ds(tok0, ch)], hd_v)

            def group_body(g, carry2):
                s = g * 16
                px = px_v[pl.ds(s, 16)]
                py = py_v[pl.ds(s, 16)]
                hd = hd_v[pl.ds(s, 16)]
                ex = px + 300.0
                ix0 = jnp.clip(ex.astype(jnp.int32), 0, n0 - 1)
                rx = ex - ix0.astype(jnp.float32)
                ix1 = jnp.clip((rx / 0.01).astype(jnp.int32), 0, n1 - 1)
                ey = py + 300.0
                iy0 = jnp.clip(ey.astype(jnp.int32), 0, n0 - 1)
                ry = ey - iy0.astype(jnp.float32)
                iy1 = jnp.clip((ry / 0.01).astype(jnp.int32), 0, n1 - 1)
                eh = hd * 180.0 / jnp.pi + 180.0
                ih0 = jnp.clip((eh / 20.0).astype(jnp.int32), 0, nh0 - 1)
                rh = eh - ih0.astype(jnp.float32) * 20.0
                ih1 = jnp.clip(rh.astype(jnp.int32), 0, nh1 - 1)

                b128 = (s + lane) * 128
                gx0 = ix0 * d0
                gx1 = ix1 * d1
                gy0 = iy0 * d0
                gy1 = iy1 * d1
                gh0 = ih0 * dh
                gh1 = ih1 * dh
                for j in range(d0):
                    v = plsc.load_gather(t0_v, [gx0 + j])
                    plsc.store_scatter(xb_v, [b128 + j], v)
                for j in range(d1):
                    v = plsc.load_gather(t1_v, [gx1 + j])
                    plsc.store_scatter(xb_v, [b128 + (d0 + j)], v)
                for j in range(d0):
                    v = plsc.load_gather(t0_v, [gy0 + j])
                    plsc.store_scatter(xb_v, [b128 + (d0 + d1 + j)], v)
                for j in range(d1):
                    v = plsc.load_gather(t1_v, [gy1 + j])
                    plsc.store_scatter(xb_v, [b128 + (2 * d0 + d1 + j)], v)
                off_h = 2 * (d0 + d1)
                for j in range(dh):
                    v = plsc.load_gather(h0_v, [gh0 + j])
                    plsc.store_scatter(xb_v, [b128 + (off_h + j)], v)
                for j in range(dh):
                    v = plsc.load_gather(h1_v, [gh1 + j])
                    plsc.store_scatter(xb_v, [b128 + (off_h + dh + j)], v)
                for j in range(off_h + 2 * dh, 128):
                    plsc.store_scatter(xb_v, [b128 + j], zeros16)
                return carry2

            lax.fori_loop(0, n_groups, group_body, 0)
            pltpu.sync_copy(xb_v, out_h.at[pl.ds(tok0 * 128, ch * 128)])
            return carry

        lax.fori_loop(0, n_chunks, chunk_body, 0)

    return sc_kernel(posx, posy, heading,
                     t0.reshape(-1), t1.reshape(-1),
                     h0.reshape(-1), h1.reshape(-1))


def _ln(x, g, b, eps=1e-5):
    mu = jnp.mean(x, axis=-1, keepdims=True)
    mu2 = jnp.mean(x * x, axis=-1, keepdims=True)
    var = jnp.maximum(mu2 - mu * mu, 0.0)
    return (x - mu) * lax.rsqrt(var + eps) * g + b


def _tc_mlp(x2d, maskf, w1p, g1, b1, w2, g2, b2, w3, b3, oob):
    tokens = x2d.shape[0]
    tile = 2048
    grid = tokens // tile
    assert grid * tile == tokens

    def body(x_ref, m_ref, w1_ref, g1_ref, b1_ref, w2_ref, g2_ref, b2_ref,
             w3_ref, b3_ref, oob_ref, o_ref):
        bf16 = jnp.bfloat16
        x = x_ref[...].astype(bf16)
        h = jnp.dot(x, w1_ref[...], preferred_element_type=jnp.float32)
        h = jnp.maximum(_ln(h, g1_ref[...], b1_ref[...]), 0.0)
        h = jnp.dot(h.astype(bf16), w2_ref[...], preferred_element_type=jnp.float32)
        h = jnp.maximum(_ln(h, g2_ref[...], b2_ref[...]), 0.0)
        y = jnp.dot(h.astype(bf16), w3_ref[...], preferred_element_type=jnp.float32)
        y = y + b3_ref[...]
        m = m_ref[...]
        o_ref[...] = jnp.where(m > 0.0, y, oob_ref[...])

    full = lambda shape: pl.BlockSpec(shape, lambda i: (0, 0))
    return pl.pallas_call(
        body,
        grid=(grid,),
        in_specs=[
            pl.BlockSpec((tile, 128), lambda i: (i, 0)),
            pl.BlockSpec((tile, 1), lambda i: (i, 0)),
            full((128, 256)),
            full((1, 256)),
            full((1, 256)),
            full((256, 256)),
            full((1, 256)),
            full((1, 256)),
            full((256, 256)),
            full((1, 256)),
            full((1, 256)),
        ],
        out_specs=pl.BlockSpec((tile, 256), lambda i: (i, 0)),
        out_shape=jax.ShapeDtypeStruct((tokens, 256), jnp.float32),
    )(x2d, maskf, w1p, g1, b1, w2, g2, b2, w3, b3, oob)


def kernel(agent_position, agent_heading, agent_valid_mask, map_polygon_center,
           map_valid_mask, pos_table_0, pos_table_1, head_table_0, head_table_1,
           w1, ln1_g, ln1_b, w2, ln2_g, ln2_b, w3, b3, oob_w, window_T):
    B, N, T = agent_heading.shape
    tokens = B * (T - 1) * N

    posx = jnp.swapaxes(agent_position[:, :, 1:, 0], 1, 2).reshape(-1)
    posy = jnp.swapaxes(agent_position[:, :, 1:, 1], 1, 2).reshape(-1)
    hd = jnp.swapaxes(agent_heading[:, :, 1:], 1, 2).reshape(-1)
    maskf = jnp.swapaxes(agent_valid_mask[:, :, 1:], 1, 2).reshape(-1, 1)
    maskf = maskf.astype(jnp.float32)

    x = _sc_features(posx, posy, hd, pos_table_0, pos_table_1,
                     head_table_0, head_table_1)
    x2d = x.reshape(tokens, 128)

    w1p = jnp.pad(w1, ((0, 128 - w1.shape[0]), (0, 0))).astype(jnp.bfloat16)
    w2 = w2.astype(jnp.bfloat16)
    w3 = w3.astype(jnp.bfloat16)
    out = _tc_mlp(x2d, maskf, w1p,
                  ln1_g.reshape(1, -1), ln1_b.reshape(1, -1),
                  w2, ln2_g.reshape(1, -1), ln2_b.reshape(1, -1),
                  w3, b3.reshape(1, -1), oob_w)
    return out.reshape(B, T - 1, N, 256)


# speedup vs baseline: 12.2364x; 2.2012x over previous
import functools

import jax
import jax.numpy as jnp
from jax import lax
from jax.experimental import pallas as pl
from jax.experimental.pallas import tpu as pltpu
from jax.experimental.pallas import tpu_sc as plsc


def _sc_features(coords, t0, t1, h0, h1):
    tokens = coords.shape[1]
    info = plsc.get_sparse_core_info()
    ncores, nsub = info.num_cores, info.num_subcores
    nw = ncores * nsub
    tpw = tokens // nw
    assert tpw * nw == tokens
    ch = 384
    assert tpw % ch == 0
    n_chunks = tpw // ch
    n_groups = ch // 16

    n0, d0 = t0.shape
    n1, d1 = t1.shape
    nh0, dh = h0.shape
    nh1, _ = h1.shape
    nfeat = 2 * (d0 + d1) + 2 * dh

    mesh = plsc.VectorSubcoreMesh(core_axis_name="c", subcore_axis_name="s")

    @functools.partial(
        pl.kernel,
        out_type=jax.ShapeDtypeStruct((nfeat, tokens), jnp.float32),
        mesh=mesh,
        compiler_params=pltpu.CompilerParams(needs_layout_passes=False),
        scratch_types=[
            pltpu.VMEM((n0 * d0,), jnp.float32),
            pltpu.VMEM((n1 * d1,), jnp.float32),
            pltpu.VMEM((nh0 * dh,), jnp.float32),
            pltpu.VMEM((nh1 * dh,), jnp.float32),
            pltpu.VMEM((3, ch), jnp.float32),
            pltpu.VMEM((nfeat, ch), jnp.float32),
        ],
    )
    def sc_kernel(co_h, t0_h, t1_h, h0_h, h1_h, out_h,
                  t0_v, t1_v, h0_v, h1_v, co_v, xb_v):
        wid = lax.axis_index("s") * ncores + lax.axis_index("c")
        base = wid * tpw
        pltpu.sync_copy(t0_h, t0_v)
        pltpu.sync_copy(t1_h, t1_v)
        pltpu.sync_copy(h0_h, h0_v)
        pltpu.sync_copy(h1_h, h1_v)

        def chunk_body(c, carry):
            tok0 = base + c * ch
            pltpu.sync_copy(co_h.at[:, pl.ds(tok0, ch)], co_v)

            def group_body(g, carry2):
                s = g * 16
                px = co_v[0, pl.ds(s, 16)]
                py = co_v[1, pl.ds(s, 16)]
                hd = co_v[2, pl.ds(s, 16)]
                ex = px + 300.0
                ix0 = jnp.clip(ex.astype(jnp.int32), 0, n0 - 1)
                rx = ex - ix0.astype(jnp.float32)
                ix1 = jnp.clip((rx / 0.01).astype(jnp.int32), 0, n1 - 1)
                ey = py + 300.0
                iy0 = jnp.clip(ey.astype(jnp.int32), 0, n0 - 1)
                ry = ey - iy0.astype(jnp.float32)
                iy1 = jnp.clip((ry / 0.01).astype(jnp.int32), 0, n1 - 1)
                eh = hd * 180.0 / jnp.pi + 180.0
                ih0 = jnp.clip((eh / 20.0).astype(jnp.int32), 0, nh0 - 1)
                rh = eh - ih0.astype(jnp.float32) * 20.0
                ih1 = jnp.clip(rh.astype(jnp.int32), 0, nh1 - 1)

                taps = [
                    (t0_v, ix0 * d0, d0, 0),
                    (t1_v, ix1 * d1, d1, d0),
                    (t0_v, iy0 * d0, d0, d0 + d1),
                    (t1_v, iy1 * d1, d1, 2 * d0 + d1),
                    (h0_v, ih0 * dh, dh, 2 * (d0 + d1)),
                    (h1_v, ih1 * dh, dh, 2 * (d0 + d1) + dh),
                ]
                items = [(tab, gidx, j, row0 + j)
                         for tab, gidx, d, row0 in taps for j in range(d)]
                for w0 in range(0, len(items), 16):
                    wave = items[w0:w0 + 16]
                    vals = [plsc.load_gather(tab, [gidx + j])
                            for tab, gidx, j, _ in wave]
                    for (_, _, _, row), v in zip(wave, vals):
                        xb_v[row, pl.ds(s, 16)] = v
                return carry2

            lax.fori_loop(0, n_groups, group_body, 0)
            pltpu.sync_copy(xb_v, out_h.at[:, pl.ds(tok0, ch)])
            return carry

        lax.fori_loop(0, n_chunks, chunk_body, 0)

    return sc_kernel(coords, t0.reshape(-1), t1.reshape(-1),
                     h0.reshape(-1), h1.reshape(-1))


def _ln(x, g, b, eps=1e-5):
    mu = jnp.mean(x, axis=-1, keepdims=True)
    mu2 = jnp.mean(x * x, axis=-1, keepdims=True)
    var = jnp.maximum(mu2 - mu * mu, 0.0)
    return (x - mu) * lax.rsqrt(var + eps) * g + b


def _tc_mlp(xt, maskf, w1, g1, b1, w2, g2, b2, w3, b3, oob):
    nfeat, tokens = xt.shape
    tile = 2048
    grid = tokens // tile
    assert grid * tile == tokens

    def body(x_ref, m_ref, w1_ref, g1_ref, b1_ref, w2_ref, g2_ref, b2_ref,
             w3_ref, b3_ref, oob_ref, o_ref):
        bf16 = jnp.bfloat16
        x = x_ref[...].astype(bf16)
        h = lax.dot_general(x, w1_ref[...], (((0,), (0,)), ((), ())),
                            preferred_element_type=jnp.float32)
        h = jnp.maximum(_ln(h, g1_ref[...], b1_ref[...]), 0.0)
        h = jnp.dot(h.astype(bf16), w2_ref[...], preferred_element_type=jnp.float32)
        h = jnp.maximum(_ln(h, g2_ref[...], b2_ref[...]), 0.0)
        y = jnp.dot(h.astype(bf16), w3_ref[...], preferred_element_type=jnp.float32)
        y = y + b3_ref[...]
        m = m_ref[...]
        o_ref[...] = jnp.where(m > 0.0, y, oob_ref[...])

    full = lambda shape: pl.BlockSpec(shape, lambda i: (0, 0))
    return pl.pallas_call(
        body,
        grid=(grid,),
        in_specs=[
            pl.BlockSpec((nfeat, tile), lambda i: (0, i)),
            pl.BlockSpec((tile, 1), lambda i: (i, 0)),
            full((nfeat, 256)),
            full((1, 256)),
            full((1, 256)),
            full((256, 256)),
            full((1, 256)),
            full((1, 256)),
            full((256, 256)),
            full((1, 256)),
            full((1, 256)),
        ],
        out_specs=pl.BlockSpec((tile, 256), lambda i: (i, 0)),
        out_shape=jax.ShapeDtypeStruct((tokens, 256), jnp.float32),
    )(xt, maskf, w1, g1, b1, w2, g2, b2, w3, b3, oob)


def kernel(agent_position, agent_heading, agent_valid_mask, map_polygon_center,
           map_valid_mask, pos_table_0, pos_table_1, head_table_0, head_table_1,
           w1, ln1_g, ln1_b, w2, ln2_g, ln2_b, w3, b3, oob_w, window_T):
    B, N, T = agent_heading.shape
    tokens = B * (T - 1) * N

    posx = jnp.swapaxes(agent_position[:, :, 1:, 0], 1, 2).reshape(-1)
    posy = jnp.swapaxes(agent_position[:, :, 1:, 1], 1, 2).reshape(-1)
    hd = jnp.swapaxes(agent_heading[:, :, 1:], 1, 2).reshape(-1)
    coords = jnp.stack([posx, posy, hd], axis=0)
    maskf = jnp.swapaxes(agent_valid_mask[:, :, 1:], 1, 2).reshape(-1, 1)
    maskf = maskf.astype(jnp.float32)

    xt = _sc_features(coords, pos_table_0, pos_table_1,
                      head_table_0, head_table_1)

    out = _tc_mlp(xt, maskf, w1.astype(jnp.bfloat16),
                  ln1_g.reshape(1, -1), ln1_b.reshape(1, -1),
                  w2.astype(jnp.bfloat16), ln2_g.reshape(1, -1),
                  ln2_b.reshape(1, -1), w3.astype(jnp.bfloat16),
                  b3.reshape(1, -1), oob_w)
    return out.reshape(B, T - 1, N, 256)


# speedup vs baseline: 13.0037x; 1.0627x over previous
import functools

import jax
import jax.numpy as jnp
from jax import lax
from jax.experimental import pallas as pl
from jax.experimental.pallas import tpu as pltpu
from jax.experimental.pallas import tpu_sc as plsc


def _sc_features(coords, t0, t1, h0, h1):
    tokens = coords.shape[1]
    info = plsc.get_sparse_core_info()
    ncores, nsub = info.num_cores, info.num_subcores
    nw = ncores * nsub
    tpw = tokens // nw
    assert tpw * nw == tokens
    ch = 640
    assert tpw % ch == 0
    n_chunks = tpw // ch
    n_groups = ch // 16

    n0, d0 = t0.shape
    n1, d1 = t1.shape
    nh0, dh = h0.shape
    nh1, _ = h1.shape
    nfeat = 2 * (d0 + d1) + 2 * dh

    mesh = plsc.VectorSubcoreMesh(core_axis_name="c", subcore_axis_name="s")

    @functools.partial(
        pl.kernel,
        out_type=jax.ShapeDtypeStruct((nfeat, tokens), jnp.float32),
        mesh=mesh,
        compiler_params=pltpu.CompilerParams(needs_layout_passes=False),
        scratch_types=[
            pltpu.VMEM((n0 * d0,), jnp.float32),
            pltpu.VMEM((n1 * d1,), jnp.float32),
            pltpu.VMEM((nh0 * dh,), jnp.float32),
            pltpu.VMEM((nh1 * dh,), jnp.float32),
            pltpu.VMEM((3, ch), jnp.float32),
            pltpu.VMEM((nfeat, ch), jnp.float32),
        ],
    )
    def sc_kernel(co_h, t0_h, t1_h, h0_h, h1_h, out_h,
                  t0_v, t1_v, h0_v, h1_v, co_v, xb_v):
        wid = lax.axis_index("s") * ncores + lax.axis_index("c")
        base = wid * tpw
        pltpu.sync_copy(t0_h, t0_v)
        pltpu.sync_copy(t1_h, t1_v)
        pltpu.sync_copy(h0_h, h0_v)
        pltpu.sync_copy(h1_h, h1_v)

        def chunk_body(c, carry):
            tok0 = base + c * ch
            pltpu.sync_copy(co_h.at[:, pl.ds(tok0, ch)], co_v)

            def group_body(g, carry2):
                s = g * 16
                px = co_v[0, pl.ds(s, 16)]
                py = co_v[1, pl.ds(s, 16)]
                hd = co_v[2, pl.ds(s, 16)]
                ex = px + 300.0
                ix0 = jnp.clip(ex.astype(jnp.int32), 0, n0 - 1)
                rx = ex - ix0.astype(jnp.float32)
                ix1 = jnp.clip((rx / 0.01).astype(jnp.int32), 0, n1 - 1)
                ey = py + 300.0
                iy0 = jnp.clip(ey.astype(jnp.int32), 0, n0 - 1)
                ry = ey - iy0.astype(jnp.float32)
                iy1 = jnp.clip((ry / 0.01).astype(jnp.int32), 0, n1 - 1)
                eh = hd * 180.0 / jnp.pi + 180.0
                ih0 = jnp.clip((eh / 20.0).astype(jnp.int32), 0, nh0 - 1)
                rh = eh - ih0.astype(jnp.float32) * 20.0
                ih1 = jnp.clip(rh.astype(jnp.int32), 0, nh1 - 1)

                taps = [
                    (t0_v, ix0 * d0, d0, 0),
                    (t1_v, ix1 * d1, d1, d0),
                    (t0_v, iy0 * d0, d0, d0 + d1),
                    (t1_v, iy1 * d1, d1, 2 * d0 + d1),
                    (h0_v, ih0 * dh, dh, 2 * (d0 + d1)),
                    (h1_v, ih1 * dh, dh, 2 * (d0 + d1) + dh),
                ]
                items = [(tab, gidx, j, row0 + j)
                         for tab, gidx, d, row0 in taps for j in range(d)]
                for w0 in range(0, len(items), 16):
                    wave = items[w0:w0 + 16]
                    vals = [plsc.load_gather(tab, [gidx + j])
                            for tab, gidx, j, _ in wave]
                    for (_, _, _, row), v in zip(wave, vals):
                        xb_v[row, pl.ds(s, 16)] = v
                return carry2

            lax.fori_loop(0, n_groups, group_body, 0)
            pltpu.sync_copy(xb_v, out_h.at[:, pl.ds(tok0, ch)])
            return carry

        lax.fori_loop(0, n_chunks, chunk_body, 0)

    return sc_kernel(coords, t0.reshape(-1), t1.reshape(-1),
                     h0.reshape(-1), h1.reshape(-1))


def _ln_relu_bf16(x, g, b, eps=1e-5):
    bf16 = jnp.bfloat16
    mu = jnp.mean(x, axis=-1, keepdims=True)
    mu2 = jnp.mean(x * x, axis=-1, keepdims=True)
    var = jnp.maximum(mu2 - mu * mu, 0.0)
    r = lax.rsqrt(var + eps)
    xb = x.astype(bf16)
    y = (xb - mu.astype(bf16)) * r.astype(bf16) * g + b
    return jnp.maximum(y, jnp.zeros((), bf16))


def _tc_mlp(xt, maskf, w1, g1, b1, w2, g2, b2, w3, b3, oob):
    nfeat, tokens = xt.shape
    tile = 2048
    grid = tokens // tile
    assert grid * tile == tokens

    def body(x_ref, m_ref, w1_ref, g1_ref, b1_ref, w2_ref, g2_ref, b2_ref,
             w3_ref, b3_ref, oob_ref, o_ref):
        bf16 = jnp.bfloat16
        x = x_ref[...].astype(bf16)
        h = lax.dot_general(x, w1_ref[...], (((0,), (0,)), ((), ())),
                            preferred_element_type=jnp.float32)
        h = _ln_relu_bf16(h, g1_ref[...], b1_ref[...])
        h = jnp.dot(h, w2_ref[...], preferred_element_type=jnp.float32)
        h = _ln_relu_bf16(h, g2_ref[...], b2_ref[...])
        y = jnp.dot(h, w3_ref[...], preferred_element_type=jnp.float32)
        y = y + b3_ref[...]
        m = m_ref[...]
        o_ref[...] = jnp.where(m > 0.0, y, oob_ref[...])

    full = lambda shape: pl.BlockSpec(shape, lambda i: (0, 0))
    return pl.pallas_call(
        body,
        grid=(grid,),
        in_specs=[
            pl.BlockSpec((nfeat, tile), lambda i: (0, i)),
            pl.BlockSpec((tile, 1), lambda i: (i, 0)),
            full((nfeat, 256)),
            full((1, 256)),
            full((1, 256)),
            full((256, 256)),
            full((1, 256)),
            full((1, 256)),
            full((256, 256)),
            full((1, 256)),
            full((1, 256)),
        ],
        out_specs=pl.BlockSpec((tile, 256), lambda i: (i, 0)),
        out_shape=jax.ShapeDtypeStruct((tokens, 256), jnp.float32),
    )(xt, maskf, w1, g1, b1, w2, g2, b2, w3, b3, oob)


def kernel(agent_position, agent_heading, agent_valid_mask, map_polygon_center,
           map_valid_mask, pos_table_0, pos_table_1, head_table_0, head_table_1,
           w1, ln1_g, ln1_b, w2, ln2_g, ln2_b, w3, b3, oob_w, window_T):
    B, N, T = agent_heading.shape
    tokens = B * (T - 1) * N

    posx = jnp.swapaxes(agent_position[:, :, 1:, 0], 1, 2).reshape(-1)
    posy = jnp.swapaxes(agent_position[:, :, 1:, 1], 1, 2).reshape(-1)
    hd = jnp.swapaxes(agent_heading[:, :, 1:], 1, 2).reshape(-1)
    coords = jnp.stack([posx, posy, hd], axis=0)
    maskf = jnp.swapaxes(agent_valid_mask[:, :, 1:], 1, 2).reshape(-1, 1)
    maskf = maskf.astype(jnp.float32)

    xt = _sc_features(coords, pos_table_0, pos_table_1,
                      head_table_0, head_table_1)

    bf16 = jnp.bfloat16
    out = _tc_mlp(xt, maskf, w1.astype(bf16),
                  ln1_g.reshape(1, -1).astype(bf16),
                  ln1_b.reshape(1, -1).astype(bf16),
                  w2.astype(bf16),
                  ln2_g.reshape(1, -1).astype(bf16),
                  ln2_b.reshape(1, -1).astype(bf16),
                  w3.astype(bf16), b3.reshape(1, -1), oob_w)
    return out.reshape(B, T - 1, N, 256)
